# baseline (speedup 1.0000x reference)
import jax
import jax.numpy as jnp
from jax import lax
from jax.experimental import pallas as pl
from jax.experimental.pallas import tpu as pltpu
from jax.experimental.pallas import tpu_sc as plsc

_NC = 2
_NS = 16
_NW = _NC * _NS

_DIM = 64
_LANE = 16
_NBUF = 4


def _make_emb(S: int, B: int, V: int):
    BB = B // _NW
    assert BB * _NW == B and BB == 128 and _DIM == 64 and S % _NBUF == 0

    mesh = plsc.VectorSubcoreMesh(core_axis_name="c", subcore_axis_name="s")

    def body(ids_hbm, w2_hbm, out_hbm, ids_v, *rest):
        jdx = rest[0:_NBUF]
        rows = rest[_NBUF:2 * _NBUF]
        obuf = rest[2 * _NBUF:3 * _NBUF]
        sg = rest[3 * _NBUF:4 * _NBUF]
        so = rest[4 * _NBUF:5 * _NBUF]

        wid = lax.axis_index("s") * _NC + lax.axis_index("c")
        b0 = wid * BB
        iota = lax.iota(jnp.int32, _LANE)

        pltpu.sync_copy(ids_hbm.at[:, pl.ds(b0, BB)], ids_v)

        def prep(s, jb):
            @pl.loop(0, BB // _LANE, unroll=True)
            def _jprep(k):
                iv = ids_v[s, pl.ds(k * _LANE, _LANE)]
                jb[pl.ds(k * _LANE, _LANE)] = iv

        def transpose(s, rb, ob):
            zero = iota * 0
            bases = [(iota + k * _LANE) * _DIM for k in range(BB // _LANE)]

            rowv = [iota + k * _LANE for k in range(BB // _LANE)]

            @plsc.parallel_loop(0, _DIM, unroll=2)
            def _dcol(c):
                for k in range(BB // _LANE):
                    dv = (rowv[k] + c) & (_DIM - 1)
                    vals = plsc.load_gather(rb, [zero, bases[k] + dv])
                    plsc.store_scatter(ob, [dv, rowv[k]], vals)

        _NSPLIT = 4
        _CH = BB // _NSPLIT

        def fire_gather(jb, rb, sem):
            for q in range(_NSPLIT):
                pltpu.async_copy(
                    w2_hbm.at[jb.at[pl.ds(q * _CH, _CH)]],
                    rb.at[pl.ds(q * _CH, _CH), :], sem)

        def wait_gather(jb, rb, sem):
            for q in range(_NSPLIT):
                pltpu.make_async_copy(
                    w2_hbm.at[jb.at[pl.ds(q * _CH, _CH)]],
                    rb.at[pl.ds(q * _CH, _CH), :], sem).wait()

        for p in range(_NBUF):
            prep(p, jdx[p])
            fire_gather(jdx[p], rows[p], sg[p])

        @pl.loop(0, S, step=_NBUF)
        def _outer(s0):
            for b in range(_NBUF):
                s = s0 + b

                wait_gather(jdx[b], rows[b], sg[b])

                @pl.when(s >= _NBUF)
                def _():
                    pltpu.make_async_copy(
                        obuf[b], out_hbm.at[s - _NBUF, :, pl.ds(b0, BB)], so[b]
                    ).wait()

                transpose(s, rows[b], obuf[b])
                pltpu.async_copy(
                    obuf[b], out_hbm.at[s, :, pl.ds(b0, BB)], so[b]
                )

                nxt = s + _NBUF

                @pl.when(nxt < S)
                def _():
                    prep(nxt, jdx[b])
                    fire_gather(jdx[b], rows[b], sg[b])

        for b in range(_NBUF):
            pltpu.make_async_copy(
                obuf[b], out_hbm.at[S - _NBUF + b, :, pl.ds(b0, BB)], so[b]
            ).wait()

    return pl.kernel(
        body,
        out_type=jax.ShapeDtypeStruct((S, _DIM, B), jnp.float32),
        mesh=mesh,
        compiler_params=pltpu.CompilerParams(
            use_tc_tiling_on_sc=False, needs_layout_passes=False
        ),
        scratch_types=(
            [pltpu.VMEM((S, BB), jnp.int32)]
            + [pltpu.VMEM((BB,), jnp.int32) for _ in range(_NBUF)]
            + [pltpu.VMEM((BB, _DIM), jnp.float32) for _ in range(_NBUF)]
            + [pltpu.VMEM((_DIM, BB), jnp.float32) for _ in range(_NBUF)]
            + [pltpu.SemaphoreType.DMA for _ in range(2 * _NBUF)]
        ),
    )


def kernel(token_ids, weight):
    B, S = token_ids.shape
    V, D = weight.shape
    ids_t = jnp.transpose(token_ids).astype(jnp.int32)
    out_p = _make_emb(S, B, V)(ids_t, weight)
    return jnp.transpose(out_p, (2, 0, 1))


# speedup vs baseline: 1.2048x; 1.2048x over previous
import jax
import jax.numpy as jnp
from jax import lax
from jax.experimental import pallas as pl
from jax.experimental.pallas import tpu as pltpu
from jax.experimental.pallas import tpu_sc as plsc

_NC = 2
_NS = 16
_NW = _NC * _NS

_DIM = 64
_LANE = 16
_NBUF = 4


def _make_emb(S: int, B: int, V: int):
    BB = B // _NW
    assert BB * _NW == B and BB == 128 and _DIM == 64 and S % _NBUF == 0

    mesh = plsc.VectorSubcoreMesh(core_axis_name="c", subcore_axis_name="s")

    def body(ids_hbm, w2_hbm, out_hbm, ids_v, *rest):
        jdx = rest[0:_NBUF]
        rows = rest[_NBUF:2 * _NBUF]
        obuf = rest[2 * _NBUF:3 * _NBUF]
        sg = rest[3 * _NBUF:4 * _NBUF]
        so = rest[4 * _NBUF:5 * _NBUF]

        wid = lax.axis_index("s") * _NC + lax.axis_index("c")
        b0 = wid * BB
        iota = lax.iota(jnp.int32, _LANE)

        pltpu.sync_copy(ids_hbm.at[:, pl.ds(b0, BB)], ids_v)

        def prep(s, jb):
            @pl.loop(0, BB // _LANE, unroll=True)
            def _jprep(k):
                iv = ids_v[s, pl.ds(k * _LANE, _LANE)]
                jb[pl.ds(k * _LANE, _LANE)] = lax.shift_right_logical(iv, 1)

        def transpose(s, rb, ob):
            zero = iota * 0
            bases = []
            for k in range(BB // _LANE):
                iv = ids_v[s, pl.ds(k * _LANE, _LANE)]
                bases.append((iota + k * _LANE) * 128 + (iv & 1) * _DIM)

            rowv = [iota + k * _LANE for k in range(BB // _LANE)]

            @plsc.parallel_loop(0, _DIM, unroll=2)
            def _dcol(c):
                for k in range(BB // _LANE):
                    dv = (rowv[k] + c) & (_DIM - 1)
                    vals = plsc.load_gather(rb, [zero, bases[k] + dv])
                    plsc.store_scatter(ob, [dv, rowv[k]], vals)

        _NSPLIT = 4
        _CH = BB // _NSPLIT

        def fire_gather(jb, rb, sem):
            for q in range(_NSPLIT):
                pltpu.async_copy(
                    w2_hbm.at[jb.at[pl.ds(q * _CH, _CH)]],
                    rb.at[pl.ds(q * _CH, _CH), :], sem)

        def wait_gather(jb, rb, sem):
            for q in range(_NSPLIT):
                pltpu.make_async_copy(
                    w2_hbm.at[jb.at[pl.ds(q * _CH, _CH)]],
                    rb.at[pl.ds(q * _CH, _CH), :], sem).wait()

        for p in range(_NBUF):
            prep(p, jdx[p])
            fire_gather(jdx[p], rows[p], sg[p])

        @pl.loop(0, S, step=_NBUF)
        def _outer(s0):
            for b in range(_NBUF):
                s = s0 + b

                wait_gather(jdx[b], rows[b], sg[b])

                @pl.when(s >= _NBUF)
                def _():
                    pltpu.make_async_copy(
                        obuf[b], out_hbm.at[s - _NBUF, :, pl.ds(b0, BB)], so[b]
                    ).wait()

                transpose(s, rows[b], obuf[b])
                pltpu.async_copy(
                    obuf[b], out_hbm.at[s, :, pl.ds(b0, BB)], so[b]
                )

                nxt = s + _NBUF

                @pl.when(nxt < S)
                def _():
                    prep(nxt, jdx[b])
                    fire_gather(jdx[b], rows[b], sg[b])

        for b in range(_NBUF):
            pltpu.make_async_copy(
                obuf[b], out_hbm.at[S - _NBUF + b, :, pl.ds(b0, BB)], so[b]
            ).wait()

    return pl.kernel(
        body,
        out_type=jax.ShapeDtypeStruct((S, _DIM, B), jnp.float32),
        mesh=mesh,
        compiler_params=pltpu.CompilerParams(
            use_tc_tiling_on_sc=True, needs_layout_passes=False
        ),
        scratch_types=(
            [pltpu.VMEM((S, BB), jnp.int32)]
            + [pltpu.VMEM((BB,), jnp.int32) for _ in range(_NBUF)]
            + [pltpu.VMEM((BB, 128), jnp.float32) for _ in range(_NBUF)]
            + [pltpu.VMEM((_DIM, BB), jnp.float32) for _ in range(_NBUF)]
            + [pltpu.SemaphoreType.DMA for _ in range(2 * _NBUF)]
        ),
    )


def kernel(token_ids, weight):
    B, S = token_ids.shape
    V, D = weight.shape
    ids_t = jnp.transpose(token_ids).astype(jnp.int32)
    w2 = weight.reshape(V // 2, 2 * D)
    out_p = _make_emb(S, B, V)(ids_t, w2)
    return jnp.transpose(out_p, (2, 0, 1))
